# Initial kernel scaffold; baseline (speedup 1.0000x reference)
import jax
import jax.numpy as jnp
from jax.experimental import pallas as pl


def kernel(x, edge_index, W1, b1, W2, b2, W3, b3, W4, b4):
    raise NotImplementedError("write your pallas kernel here")


# speedup vs baseline: 7.3454x; 7.3454x over previous
import functools

import jax
import jax.numpy as jnp
from jax import lax
from jax.experimental import pallas as pl
from jax.experimental.pallas import tpu as pltpu
from jax.experimental.pallas import tpu_sc as plsc

_NC = 2
_NS = 16
_NW = _NC * _NS
_L = 16
_EB = 400


def _row_block(n):
    for r in (2000, 2500, 1000, 800, 500, 400, 250, 200, 125, 100, 50, 25, 8, 5, 4, 2, 1):
        if n % r == 0:
            return r
    return 1


@functools.cache
def _sc_agg_fn(C, N, E):
    ncap = ((N + 1 + 127) // 128) * 128
    nslice = ncap // _NS
    epw = E // _NW
    nblk = epw // _EB
    assert epw % _EB == 0 and E % _NW == 0
    zch = []
    off = 0
    while off < nslice:
        sz = min(_EB, nslice - off)
        zch.append((off, sz))
        off += sz

    mesh = plsc.VectorSubcoreMesh(core_axis_name="c", subcore_axis_name="s")

    @functools.partial(
        pl.kernel,
        out_type=jax.ShapeDtypeStruct((_NC, C, ncap, _L), jnp.float32),
        mesh=mesh,
        scratch_types=[
            pltpu.VMEM_SHARED((ncap, _L), jnp.float32),
            pltpu.VMEM((_EB,), jnp.int32),
            pltpu.VMEM((_EB,), jnp.int32),
            pltpu.VMEM((_EB, _L), jnp.float32),
            pltpu.SemaphoreType.DMA,
        ],
        compiler_params=pltpu.CompilerParams(use_tc_tiling_on_sc=False),
    )
    def agg(h_ref, src_ref, dst_ref, out_ref, acc, srcb, dstb, rowb, sem):
        cid = lax.axis_index("c")
        sid = lax.axis_index("s")
        wid = sid * _NC + cid
        base = sid * nslice

        for c in range(C):
            def _zero(i, carry):
                rowb[i, :] = jnp.zeros((_L,), jnp.float32)
                return carry

            lax.fori_loop(0, _EB, _zero, 0)
            for zoff, zsz in zch:
                pltpu.sync_copy(rowb.at[pl.ds(0, zsz)], acc.at[pl.ds(base + zoff, zsz)])
            plsc.subcore_barrier()

            def _blk(j, carry):
                e0 = wid * epw + j * _EB
                pltpu.sync_copy(src_ref.at[pl.ds(e0, _EB)], srcb)
                pltpu.sync_copy(dst_ref.at[pl.ds(e0, _EB)], dstb)
                if c > 0:
                    def _addo(i, cc):
                        srcb[pl.ds(i * _L, _L)] = srcb[pl.ds(i * _L, _L)] + (c * N)
                        return cc

                    lax.fori_loop(0, _EB // _L, _addo, 0)
                pltpu.async_copy(h_ref.at[srcb], rowb, sem).wait()
                pltpu.sync_copy(rowb, acc.at[dstb], add=True)
                return carry

            lax.fori_loop(0, nblk, _blk, 0)
            plsc.subcore_barrier()
            pltpu.sync_copy(acc.at[pl.ds(base, nslice)],
                            out_ref.at[cid, c, pl.ds(base, nslice)])

    return agg


@functools.cache
def _tc_layer_fn(C, Cp, N, kind):
    ncap = ((N + 1 + 127) // 128) * 128
    R = _row_block(N)
    grid = (N // R,)
    din, dout = C * _L, Cp * _L

    in_specs = [
        pl.BlockSpec((_NC, C, R, _L), lambda i: (0, 0, i, 0)),
        pl.BlockSpec((C, R, _L), lambda i: (0, i, 0)),
    ]
    if kind != "first":
        in_specs.append(pl.BlockSpec((R, 1), lambda i: (i, 0)))
    in_specs.append(pl.BlockSpec((din, dout), lambda i: (0, 0)))
    in_specs.append(pl.BlockSpec((1, dout), lambda i: (0, 0)))

    if kind == "first":
        out_shape = [
            jax.ShapeDtypeStruct((Cp, N, _L), jnp.float32),
            jax.ShapeDtypeStruct((N, 1), jnp.float32),
        ]
        out_specs = [
            pl.BlockSpec((Cp, R, _L), lambda i: (0, i, 0)),
            pl.BlockSpec((R, 1), lambda i: (i, 0)),
        ]
    elif kind == "mid":
        out_shape = jax.ShapeDtypeStruct((Cp, N, _L), jnp.float32)
        out_specs = pl.BlockSpec((Cp, R, _L), lambda i: (0, i, 0))
    else:
        out_shape = jax.ShapeDtypeStruct((1, dout), jnp.float32)
        out_specs = pl.BlockSpec((1, dout), lambda i: (0, 0))

    def body(*refs):
        if kind == "first":
            parts, h, w, b = refs[:4]
            outs = refs[4:]
        else:
            parts, h, dinv_ref, w, b = refs[:5]
            outs = refs[5:]
        t = [parts[0, c] + parts[1, c] + h[c] for c in range(C)]
        if kind == "first":
            dv = 1.0 / t[0][:, 6:7]
            outs[1][...] = dv
        else:
            dv = dinv_ref[...]
        ochunks = []
        for cp in range(Cp):
            acc = jnp.zeros((R, _L), jnp.float32)
            for c in range(C):
                acc = acc + jnp.dot(
                    t[c], w[c * _L:(c + 1) * _L, cp * _L:(cp + 1) * _L],
                    preferred_element_type=jnp.float32)
            o = jnp.maximum(acc * dv + b[0:1, cp * _L:(cp + 1) * _L], 0.0)
            if kind == "last":
                ochunks.append(o)
            else:
                outs[0][cp] = o
        if kind == "last":
            s = jnp.sum(jnp.concatenate(ochunks, axis=1), axis=0, keepdims=True)
            pid = pl.program_id(0)

            @pl.when(pid == 0)
            def _():
                outs[0][...] = jnp.zeros((1, dout), jnp.float32)

            outs[0][...] += s

            @pl.when(pid == grid[0] - 1)
            def _():
                outs[0][...] = outs[0][...] * (1.0 / N)

    return pl.pallas_call(
        body,
        grid=grid,
        in_specs=in_specs,
        out_specs=out_specs,
        out_shape=out_shape,
        compiler_params=pltpu.CompilerParams(
            dimension_semantics=("arbitrary",)),
    )


def kernel(x, edge_index, W1, b1, W2, b2, W3, b3, W4, b4):
    N, d0 = x.shape
    ei = jnp.asarray(edge_index, jnp.int32)
    src, dst = ei[0], ei[1]
    E = int(src.shape[0])
    epad = -(-E // (_NW * _EB)) * (_NW * _EB)
    if epad != E:
        src = jnp.concatenate([src, jnp.zeros((epad - E,), jnp.int32)])
        dst = jnp.concatenate([dst, jnp.full((epad - E,), N, jnp.int32)])

    x_pad = jnp.concatenate(
        [x.astype(jnp.float32),
         jnp.ones((N, 1), jnp.float32),
         jnp.zeros((N, _L - d0 - 1), jnp.float32)], axis=1)
    w1p = jnp.zeros((_L, W1.shape[1]), jnp.float32).at[:d0].set(W1)

    h = x_pad
    parts = _sc_agg_fn(1, N, epad)(h, src, dst)
    h, dinv = _tc_layer_fn(1, 1, N, "first")(
        parts, h.reshape(1, N, _L), w1p, b1.reshape(1, -1))

    parts = _sc_agg_fn(1, N, epad)(h.reshape(N, _L), src, dst)
    h = _tc_layer_fn(1, 2, N, "mid")(
        parts, h, dinv, W2, b2.reshape(1, -1))

    parts = _sc_agg_fn(2, N, epad)(h.reshape(2 * N, _L), src, dst)
    h = _tc_layer_fn(2, 4, N, "mid")(
        parts, h, dinv, W3, b3.reshape(1, -1))

    parts = _sc_agg_fn(4, N, epad)(h.reshape(4 * N, _L), src, dst)
    out = _tc_layer_fn(4, 8, N, "last")(
        parts, h, dinv, W4, b4.reshape(1, -1))
    return out.reshape(-1)


# speedup vs baseline: 9.9175x; 1.3502x over previous
import functools

import jax
import jax.numpy as jnp
from jax import lax
from jax.experimental import pallas as pl
from jax.experimental.pallas import tpu as pltpu
from jax.experimental.pallas import tpu_sc as plsc

_NC = 2
_NS = 16
_NW = _NC * _NS
_L = 16
_EB = 400


def _row_block(n):
    for r in (2000, 2500, 1000, 800, 500, 400, 250, 200, 125, 100, 50, 25, 8, 5, 4, 2, 1):
        if n % r == 0:
            return r
    return 1


@functools.cache
def _sc_agg_fn(C, N, E):
    ncap = ((N + 1 + 127) // 128) * 128
    nslice = ncap // _NS
    epw = E // _NW
    nblk = epw // _EB
    assert epw % _EB == 0 and E % _NW == 0
    zch = []
    off = 0
    while off < nslice:
        sz = min(_EB, nslice - off)
        zch.append((off, sz))
        off += sz

    mesh = plsc.VectorSubcoreMesh(core_axis_name="c", subcore_axis_name="s")

    @functools.partial(
        pl.kernel,
        out_type=jax.ShapeDtypeStruct((_NC, C, ncap, _L), jnp.float32),
        mesh=mesh,
        scratch_types=[
            pltpu.VMEM_SHARED((ncap, _L), jnp.float32),
            pltpu.VMEM((_EB,), jnp.int32),
            pltpu.VMEM((_EB,), jnp.int32),
            pltpu.VMEM((_EB, _L), jnp.float32),
            pltpu.VMEM((_EB,), jnp.int32),
            pltpu.VMEM((_EB,), jnp.int32),
            pltpu.VMEM((_EB, _L), jnp.float32),
            pltpu.SemaphoreType.DMA,
            pltpu.SemaphoreType.DMA,
        ],
        compiler_params=pltpu.CompilerParams(use_tc_tiling_on_sc=False),
    )
    def agg(h_ref, src_ref, dst_ref, out_ref, acc,
            srcb0, dstb0, rowb0, srcb1, dstb1, rowb1, sem0, sem1):
        cid = lax.axis_index("c")
        sid = lax.axis_index("s")
        wid = sid * _NC + cid
        base = sid * nslice
        bufs = ((srcb0, dstb0, rowb0, sem0), (srcb1, dstb1, rowb1, sem1))

        for c in range(C):
            def _stage(j, buf):
                srcb, dstb, rowb, sem = buf
                e0 = wid * epw + j * _EB
                pltpu.sync_copy(src_ref.at[pl.ds(e0, _EB)], srcb)
                pltpu.sync_copy(dst_ref.at[pl.ds(e0, _EB)], dstb)
                if c > 0:
                    def _addo(i, cc):
                        srcb[pl.ds(i * _L, _L)] = srcb[pl.ds(i * _L, _L)] + (c * N)
                        return cc

                    lax.fori_loop(0, _EB // _L, _addo, 0)
                pltpu.async_copy(h_ref.at[srcb], rowb, sem)

            def _drain(buf):
                srcb, dstb, rowb, sem = buf
                pltpu.make_async_copy(h_ref.at[srcb], rowb, sem).wait()
                pltpu.sync_copy(rowb, acc.at[dstb], add=True)

            def _zero(i, carry):
                rowb0[i, :] = jnp.zeros((_L,), jnp.float32)
                return carry

            lax.fori_loop(0, _EB, _zero, 0)
            for zoff, zsz in zch:
                pltpu.sync_copy(rowb0.at[pl.ds(0, zsz)], acc.at[pl.ds(base + zoff, zsz)])
            plsc.subcore_barrier()

            _stage(0, bufs[0])

            def _pair(k, carry):
                j = 2 * k
                _stage(j + 1, bufs[1])
                _drain(bufs[0])
                _stage(j + 2, bufs[0])
                _drain(bufs[1])
                return carry

            lax.fori_loop(0, (nblk - 1) // 2, _pair, 0)
            _drain(bufs[0])
            plsc.subcore_barrier()
            pltpu.sync_copy(acc.at[pl.ds(base, nslice)],
                            out_ref.at[cid, c, pl.ds(base, nslice)])

    return agg


@functools.cache
def _tc_layer_fn(C, Cp, N, kind):
    ncap = ((N + 1 + 127) // 128) * 128
    R = _row_block(N)
    grid = (N // R,)
    din, dout = C * _L, Cp * _L

    in_specs = [
        pl.BlockSpec((_NC, C, R, _L), lambda i: (0, 0, i, 0)),
        pl.BlockSpec((C, R, _L), lambda i: (0, i, 0)),
    ]
    if kind != "first":
        in_specs.append(pl.BlockSpec((R, 1), lambda i: (i, 0)))
    in_specs.append(pl.BlockSpec((din, dout), lambda i: (0, 0)))
    in_specs.append(pl.BlockSpec((1, dout), lambda i: (0, 0)))

    if kind == "first":
        out_shape = [
            jax.ShapeDtypeStruct((Cp, N, _L), jnp.float32),
            jax.ShapeDtypeStruct((N, 1), jnp.float32),
        ]
        out_specs = [
            pl.BlockSpec((Cp, R, _L), lambda i: (0, i, 0)),
            pl.BlockSpec((R, 1), lambda i: (i, 0)),
        ]
    elif kind == "mid":
        out_shape = jax.ShapeDtypeStruct((Cp, N, _L), jnp.float32)
        out_specs = pl.BlockSpec((Cp, R, _L), lambda i: (0, i, 0))
    else:
        out_shape = jax.ShapeDtypeStruct((1, dout), jnp.float32)
        out_specs = pl.BlockSpec((1, dout), lambda i: (0, 0))

    def body(*refs):
        if kind == "first":
            parts, h, w, b = refs[:4]
            outs = refs[4:]
        else:
            parts, h, dinv_ref, w, b = refs[:5]
            outs = refs[5:]
        t = [parts[0, c] + parts[1, c] + h[c] for c in range(C)]
        if kind == "first":
            dv = 1.0 / t[0][:, 6:7]
            outs[1][...] = dv
        else:
            dv = dinv_ref[...]
        ochunks = []
        for cp in range(Cp):
            acc = jnp.zeros((R, _L), jnp.float32)
            for c in range(C):
                acc = acc + jnp.dot(
                    t[c], w[c * _L:(c + 1) * _L, cp * _L:(cp + 1) * _L],
                    preferred_element_type=jnp.float32)
            o = jnp.maximum(acc * dv + b[0:1, cp * _L:(cp + 1) * _L], 0.0)
            if kind == "last":
                ochunks.append(o)
            else:
                outs[0][cp] = o
        if kind == "last":
            s = jnp.sum(jnp.concatenate(ochunks, axis=1), axis=0, keepdims=True)
            pid = pl.program_id(0)

            @pl.when(pid == 0)
            def _():
                outs[0][...] = jnp.zeros((1, dout), jnp.float32)

            outs[0][...] += s

            @pl.when(pid == grid[0] - 1)
            def _():
                outs[0][...] = outs[0][...] * (1.0 / N)

    return pl.pallas_call(
        body,
        grid=grid,
        in_specs=in_specs,
        out_specs=out_specs,
        out_shape=out_shape,
        compiler_params=pltpu.CompilerParams(
            dimension_semantics=("arbitrary",)),
    )


def kernel(x, edge_index, W1, b1, W2, b2, W3, b3, W4, b4):
    N, d0 = x.shape
    ei = jnp.asarray(edge_index, jnp.int32)
    src, dst = ei[0], ei[1]
    E = int(src.shape[0])
    epad = -(-E // (_NW * _EB)) * (_NW * _EB)
    if epad != E:
        src = jnp.concatenate([src, jnp.zeros((epad - E,), jnp.int32)])
        dst = jnp.concatenate([dst, jnp.full((epad - E,), N, jnp.int32)])

    x_pad = jnp.concatenate(
        [x.astype(jnp.float32),
         jnp.ones((N, 1), jnp.float32),
         jnp.zeros((N, _L - d0 - 1), jnp.float32)], axis=1)
    w1p = jnp.zeros((_L, W1.shape[1]), jnp.float32).at[:d0].set(W1)

    h = x_pad
    parts = _sc_agg_fn(1, N, epad)(h, src, dst)
    h, dinv = _tc_layer_fn(1, 1, N, "first")(
        parts, h.reshape(1, N, _L), w1p, b1.reshape(1, -1))

    parts = _sc_agg_fn(1, N, epad)(h.reshape(N, _L), src, dst)
    h = _tc_layer_fn(1, 2, N, "mid")(
        parts, h, dinv, W2, b2.reshape(1, -1))

    parts = _sc_agg_fn(2, N, epad)(h.reshape(2 * N, _L), src, dst)
    h = _tc_layer_fn(2, 4, N, "mid")(
        parts, h, dinv, W3, b3.reshape(1, -1))

    parts = _sc_agg_fn(4, N, epad)(h.reshape(4 * N, _L), src, dst)
    out = _tc_layer_fn(4, 8, N, "last")(
        parts, h, dinv, W4, b4.reshape(1, -1))
    return out.reshape(-1)


# speedup vs baseline: 10.7670x; 1.0857x over previous
import functools

import jax
import jax.numpy as jnp
from jax import lax
from jax.experimental import pallas as pl
from jax.experimental.pallas import tpu as pltpu
from jax.experimental.pallas import tpu_sc as plsc

_NC = 2
_NS = 16
_NW = _NC * _NS
_L = 16
_EB = 400


def _row_block(n):
    for r in (2000, 2500, 1000, 800, 500, 400, 250, 200, 125, 100, 50, 25, 8, 5, 4, 2, 1):
        if n % r == 0:
            return r
    return 1


@functools.cache
def _sc_agg_fn(C, N, E):
    ncap = ((N + 1 + 127) // 128) * 128
    nslice = ncap // _NS
    epw = E // _NW
    nblk = epw // _EB
    assert epw % _EB == 0 and E % _NW == 0
    zch = []
    off = 0
    while off < nslice:
        sz = min(_EB, nslice - off)
        zch.append((off, sz))
        off += sz

    mesh = plsc.VectorSubcoreMesh(core_axis_name="c", subcore_axis_name="s")

    @functools.partial(
        pl.kernel,
        out_type=jax.ShapeDtypeStruct((_NC, C, ncap, _L), jnp.float32),
        mesh=mesh,
        scratch_types=[
            pltpu.VMEM_SHARED((ncap, _L), jnp.float32),
            pltpu.VMEM((_EB,), jnp.int32),
            pltpu.VMEM((_EB,), jnp.int32),
            pltpu.VMEM((_EB, _L), jnp.float32),
            pltpu.VMEM((_EB,), jnp.int32),
            pltpu.VMEM((_EB,), jnp.int32),
            pltpu.VMEM((_EB, _L), jnp.float32),
            pltpu.SemaphoreType.DMA,
            pltpu.SemaphoreType.DMA,
            pltpu.SemaphoreType.DMA,
            pltpu.SemaphoreType.DMA,
            pltpu.SemaphoreType.DMA,
            pltpu.SemaphoreType.DMA,
        ],
        compiler_params=pltpu.CompilerParams(use_tc_tiling_on_sc=False),
    )
    def agg(h_ref, src_ref, dst_ref, out_ref, acc,
            srcb0, dstb0, rowb0, srcb1, dstb1, rowb1,
            isem0, isem1, gsem0, gsem1, ssem0, ssem1):
        cid = lax.axis_index("c")
        sid = lax.axis_index("s")
        wid = sid * _NC + cid
        base = sid * nslice
        bufs = ((srcb0, dstb0, rowb0, isem0, gsem0, ssem0),
                (srcb1, dstb1, rowb1, isem1, gsem1, ssem1))

        def _idx_start(j, buf):
            srcb, dstb, _, isem, _, _ = buf
            e0 = wid * epw + j * _EB
            pltpu.async_copy(src_ref.at[pl.ds(e0, _EB)], srcb, isem)
            pltpu.async_copy(dst_ref.at[pl.ds(e0, _EB)], dstb, isem)

        def _idx_wait(buf):
            srcb, dstb, _, isem, _, _ = buf
            pltpu.make_async_copy(src_ref.at[pl.ds(0, _EB)], srcb, isem).wait()
            pltpu.make_async_copy(dst_ref.at[pl.ds(0, _EB)], dstb, isem).wait()

        def _gather_start(buf, c):
            srcb, _, rowb, _, gsem, _ = buf
            if c > 0:
                def _addo(i, cc):
                    srcb[pl.ds(i * _L, _L)] = srcb[pl.ds(i * _L, _L)] + (c * N)
                    return cc

                lax.fori_loop(0, _EB // _L, _addo, 0)
            pltpu.async_copy(h_ref.at[srcb], rowb, gsem)

        def _gather_wait(buf):
            srcb, _, rowb, _, gsem, _ = buf
            pltpu.make_async_copy(h_ref.at[srcb], rowb, gsem).wait()

        def _scatter_start(buf):
            _, dstb, rowb, _, _, ssem = buf
            pltpu.async_copy(rowb, acc.at[dstb], ssem, add=True)

        def _scatter_wait(buf):
            _, dstb, rowb, _, _, ssem = buf
            pltpu.make_async_copy(rowb, acc.at[dstb], ssem).wait()

        for c in range(C):
            def _zero(i, carry):
                rowb0[i, :] = jnp.zeros((_L,), jnp.float32)
                return carry

            lax.fori_loop(0, _EB, _zero, 0)
            for zoff, zsz in zch:
                pltpu.sync_copy(rowb0.at[pl.ds(0, zsz)], acc.at[pl.ds(base + zoff, zsz)])
            plsc.subcore_barrier()

            _idx_start(0, bufs[0])
            _idx_start(1, bufs[1])
            _idx_wait(bufs[0])
            _gather_start(bufs[0], c)

            def _pair(k, carry):
                j = 2 * k
                _gather_wait(bufs[0])
                _idx_start(j + 2, bufs[0])

                @pl.when(k > 0)
                def _():
                    _scatter_wait(bufs[1])

                _idx_wait(bufs[1])
                _gather_start(bufs[1], c)
                _scatter_start(bufs[0])

                _gather_wait(bufs[1])
                _idx_start(j + 3, bufs[1])
                _scatter_wait(bufs[0])
                _idx_wait(bufs[0])
                _gather_start(bufs[0], c)
                _scatter_start(bufs[1])
                return carry

            lax.fori_loop(0, (nblk - 1) // 2, _pair, 0)
            _gather_wait(bufs[0])
            _scatter_wait(bufs[1])
            _scatter_start(bufs[0])
            _scatter_wait(bufs[0])
            _idx_wait(bufs[1])
            plsc.subcore_barrier()
            pltpu.sync_copy(acc.at[pl.ds(base, nslice)],
                            out_ref.at[cid, c, pl.ds(base, nslice)])

    return agg


@functools.cache
def _tc_layer_fn(C, Cp, N, kind):
    ncap = ((N + 1 + 127) // 128) * 128
    R = _row_block(N)
    grid = (N // R,)
    din, dout = C * _L, Cp * _L

    in_specs = [
        pl.BlockSpec((_NC, C, R, _L), lambda i: (0, 0, i, 0)),
        pl.BlockSpec((C, R, _L), lambda i: (0, i, 0)),
    ]
    if kind != "first":
        in_specs.append(pl.BlockSpec((R, 1), lambda i: (i, 0)))
    in_specs.append(pl.BlockSpec((din, dout), lambda i: (0, 0)))
    in_specs.append(pl.BlockSpec((1, dout), lambda i: (0, 0)))

    if kind == "first":
        out_shape = [
            jax.ShapeDtypeStruct((Cp, N, _L), jnp.float32),
            jax.ShapeDtypeStruct((N, 1), jnp.float32),
        ]
        out_specs = [
            pl.BlockSpec((Cp, R, _L), lambda i: (0, i, 0)),
            pl.BlockSpec((R, 1), lambda i: (i, 0)),
        ]
    elif kind == "mid":
        out_shape = jax.ShapeDtypeStruct((Cp, N, _L), jnp.float32)
        out_specs = pl.BlockSpec((Cp, R, _L), lambda i: (0, i, 0))
    else:
        out_shape = jax.ShapeDtypeStruct((1, dout), jnp.float32)
        out_specs = pl.BlockSpec((1, dout), lambda i: (0, 0))

    def body(*refs):
        if kind == "first":
            parts, h, w, b = refs[:4]
            outs = refs[4:]
        else:
            parts, h, dinv_ref, w, b = refs[:5]
            outs = refs[5:]
        t = [parts[0, c] + parts[1, c] + h[c] for c in range(C)]
        if kind == "first":
            dv = 1.0 / t[0][:, 6:7]
            outs[1][...] = dv
        else:
            dv = dinv_ref[...]
        ochunks = []
        for cp in range(Cp):
            acc = jnp.zeros((R, _L), jnp.float32)
            for c in range(C):
                acc = acc + jnp.dot(
                    t[c], w[c * _L:(c + 1) * _L, cp * _L:(cp + 1) * _L],
                    preferred_element_type=jnp.float32)
            o = jnp.maximum(acc * dv + b[0:1, cp * _L:(cp + 1) * _L], 0.0)
            if kind == "last":
                ochunks.append(o)
            else:
                outs[0][cp] = o
        if kind == "last":
            s = jnp.sum(jnp.concatenate(ochunks, axis=1), axis=0, keepdims=True)
            pid = pl.program_id(0)

            @pl.when(pid == 0)
            def _():
                outs[0][...] = jnp.zeros((1, dout), jnp.float32)

            outs[0][...] += s

            @pl.when(pid == grid[0] - 1)
            def _():
                outs[0][...] = outs[0][...] * (1.0 / N)

    return pl.pallas_call(
        body,
        grid=grid,
        in_specs=in_specs,
        out_specs=out_specs,
        out_shape=out_shape,
        compiler_params=pltpu.CompilerParams(
            dimension_semantics=("arbitrary",)),
    )


def kernel(x, edge_index, W1, b1, W2, b2, W3, b3, W4, b4):
    N, d0 = x.shape
    ei = jnp.asarray(edge_index, jnp.int32)
    src, dst = ei[0], ei[1]
    E = int(src.shape[0])
    epad = -(-E // (_NW * _EB)) * (_NW * _EB)
    pad = epad + _EB - E
    src = jnp.concatenate([src, jnp.zeros((pad,), jnp.int32)])
    dst = jnp.concatenate([dst, jnp.full((pad,), N, jnp.int32)])

    x_pad = jnp.concatenate(
        [x.astype(jnp.float32),
         jnp.ones((N, 1), jnp.float32),
         jnp.zeros((N, _L - d0 - 1), jnp.float32)], axis=1)
    w1p = jnp.zeros((_L, W1.shape[1]), jnp.float32).at[:d0].set(W1)

    h = x_pad
    parts = _sc_agg_fn(1, N, epad)(h, src, dst)
    h, dinv = _tc_layer_fn(1, 1, N, "first")(
        parts, h.reshape(1, N, _L), w1p, b1.reshape(1, -1))

    parts = _sc_agg_fn(1, N, epad)(h.reshape(N, _L), src, dst)
    h = _tc_layer_fn(1, 2, N, "mid")(
        parts, h, dinv, W2, b2.reshape(1, -1))

    parts = _sc_agg_fn(2, N, epad)(h.reshape(2 * N, _L), src, dst)
    h = _tc_layer_fn(2, 4, N, "mid")(
        parts, h, dinv, W3, b3.reshape(1, -1))

    parts = _sc_agg_fn(4, N, epad)(h.reshape(4 * N, _L), src, dst)
    out = _tc_layer_fn(4, 8, N, "last")(
        parts, h, dinv, W4, b4.reshape(1, -1))
    return out.reshape(-1)


# speedup vs baseline: 17.9377x; 1.6660x over previous
import functools

import jax
import jax.numpy as jnp
from jax import lax
from jax.experimental import pallas as pl
from jax.experimental.pallas import tpu as pltpu
from jax.experimental.pallas import tpu_sc as plsc

_NC = 2
_NS = 16
_NW = _NC * _NS
_L = 16
_EB = 400


def _row_block(n):
    for r in (2000, 2500, 1000, 800, 500, 400, 250, 200, 125, 100, 50, 25, 8, 5, 4, 2, 1):
        if n % r == 0:
            return r
    return 1


@functools.cache
def _sc_agg_fn(C, N, NN, E):
    ncap = ((N + 1 + 127) // 128) * 128
    nslice = ncap // _NS
    epw = E // _NW
    nblk = epw // _EB
    assert epw % _EB == 0 and E % _NW == 0
    zch = []
    off = 0
    while off < nslice:
        sz = min(_EB, nslice - off)
        zch.append((off, sz))
        off += sz

    mesh = plsc.VectorSubcoreMesh(core_axis_name="c", subcore_axis_name="s")

    @functools.partial(
        pl.kernel,
        out_type=jax.ShapeDtypeStruct((_NC, C, ncap, _L), jnp.float32),
        mesh=mesh,
        scratch_types=[
            pltpu.VMEM_SHARED((ncap, _L), jnp.float32),
            pltpu.VMEM((_EB,), jnp.int32),
            pltpu.VMEM((_EB,), jnp.int32),
            pltpu.VMEM((_EB, _L), jnp.float32),
            pltpu.VMEM((_EB,), jnp.int32),
            pltpu.VMEM((_EB,), jnp.int32),
            pltpu.VMEM((_EB, _L), jnp.float32),
            pltpu.SemaphoreType.DMA,
            pltpu.SemaphoreType.DMA,
            pltpu.SemaphoreType.DMA,
            pltpu.SemaphoreType.DMA,
            pltpu.SemaphoreType.DMA,
            pltpu.SemaphoreType.DMA,
        ],
        compiler_params=pltpu.CompilerParams(use_tc_tiling_on_sc=False),
    )
    def agg(h_ref, src_ref, dst_ref, out_ref, acc,
            srcb0, dstb0, rowb0, srcb1, dstb1, rowb1,
            isem0, isem1, gsem0, gsem1, ssem0, ssem1):
        cid = lax.axis_index("c")
        sid = lax.axis_index("s")
        wid = sid * _NC + cid
        base = sid * nslice
        bufs = ((srcb0, dstb0, rowb0, isem0, gsem0, ssem0),
                (srcb1, dstb1, rowb1, isem1, gsem1, ssem1))

        def _idx_start(j, buf):
            srcb, dstb, _, isem, _, _ = buf
            e0 = wid * epw + j * _EB
            pltpu.async_copy(src_ref.at[pl.ds(e0, _EB)], srcb, isem)
            pltpu.async_copy(dst_ref.at[pl.ds(e0, _EB)], dstb, isem)

        def _idx_wait(buf):
            srcb, dstb, _, isem, _, _ = buf
            pltpu.make_async_copy(src_ref.at[pl.ds(0, _EB)], srcb, isem).wait()
            pltpu.make_async_copy(dst_ref.at[pl.ds(0, _EB)], dstb, isem).wait()

        def _gather_start(buf, c):
            srcb, _, rowb, _, gsem, _ = buf
            if c > 0:
                def _addo(i, cc):
                    srcb[pl.ds(i * _L, _L)] = srcb[pl.ds(i * _L, _L)] + (c * NN)
                    return cc

                lax.fori_loop(0, _EB // _L, _addo, 0)
            pltpu.async_copy(h_ref.at[srcb], rowb, gsem)

        def _gather_wait(buf):
            srcb, _, rowb, _, gsem, _ = buf
            pltpu.make_async_copy(h_ref.at[srcb], rowb, gsem).wait()

        def _scatter_start(buf):
            _, dstb, rowb, _, _, ssem = buf
            pltpu.async_copy(rowb, acc.at[dstb], ssem, add=True)

        def _scatter_wait(buf):
            _, dstb, rowb, _, _, ssem = buf
            pltpu.make_async_copy(rowb, acc.at[dstb], ssem).wait()

        for c in range(C):
            def _zero(i, carry):
                rowb0[i, :] = jnp.zeros((_L,), jnp.float32)
                return carry

            lax.fori_loop(0, _EB, _zero, 0)
            for zoff, zsz in zch:
                pltpu.sync_copy(rowb0.at[pl.ds(0, zsz)], acc.at[pl.ds(base + zoff, zsz)])
            plsc.subcore_barrier()

            _idx_start(0, bufs[0])
            _idx_start(1, bufs[1])
            _idx_wait(bufs[0])
            _gather_start(bufs[0], c)

            def _pair(k, carry):
                j = 2 * k
                _gather_wait(bufs[0])
                _idx_start(j + 2, bufs[0])

                @pl.when(k > 0)
                def _():
                    _scatter_wait(bufs[1])

                _idx_wait(bufs[1])
                _gather_start(bufs[1], c)
                _scatter_start(bufs[0])

                _gather_wait(bufs[1])
                _idx_start(j + 3, bufs[1])
                _scatter_wait(bufs[0])
                _idx_wait(bufs[0])
                _gather_start(bufs[0], c)
                _scatter_start(bufs[1])
                return carry

            lax.fori_loop(0, (nblk - 1) // 2, _pair, 0)
            _gather_wait(bufs[0])
            _scatter_wait(bufs[1])
            _scatter_start(bufs[0])
            _scatter_wait(bufs[0])
            _idx_wait(bufs[1])
            plsc.subcore_barrier()
            pltpu.sync_copy(acc.at[pl.ds(base, nslice)],
                            out_ref.at[cid, c, pl.ds(base, nslice)])

    return agg


@functools.cache
def _tc_layer_fn(C, Cp, N, NN, kind):
    ncap = ((N + 1 + 127) // 128) * 128
    rb = 256
    nreal = N * _L // 128
    nrow = NN * _L // 128
    grid = (-(-nreal // rb),)

    in_specs = [
        pl.BlockSpec((_NC, C, rb, 128), lambda i: (0, 0, i, 0)),
        pl.BlockSpec((C, rb, 128), lambda i: (0, i, 0)),
    ]
    if kind != "first":
        in_specs.append(pl.BlockSpec((rb, 128), lambda i: (i, 0)))
    in_specs.append(pl.BlockSpec((C, Cp, 128, 128), lambda i: (0, 0, 0, 0)))
    in_specs.append(pl.BlockSpec((Cp, 128), lambda i: (0, 0)))

    if kind == "first":
        out_shape = [
            jax.ShapeDtypeStruct((Cp, nrow, 128), jnp.float32),
            jax.ShapeDtypeStruct((nrow, 128), jnp.float32),
        ]
        out_specs = [
            pl.BlockSpec((Cp, rb, 128), lambda i: (0, i, 0)),
            pl.BlockSpec((rb, 128), lambda i: (i, 0)),
        ]
    elif kind == "mid":
        out_shape = jax.ShapeDtypeStruct((Cp, nrow, 128), jnp.float32)
        out_specs = pl.BlockSpec((Cp, rb, 128), lambda i: (0, i, 0))
    else:
        out_shape = jax.ShapeDtypeStruct((Cp, 128), jnp.float32)
        out_specs = pl.BlockSpec((Cp, 128), lambda i: (0, 0))

    def body(*refs):
        if kind == "first":
            parts, h, w, b = refs[:4]
            outs = refs[4:]
        else:
            parts, h, dinv_ref, w, b = refs[:5]
            outs = refs[5:]
        t = [parts[0, c] + parts[1, c] + h[c] for c in range(C)]
        if kind == "first":
            li = lax.broadcasted_iota(jnp.int32, (128, 128), 0)
            mi = lax.broadcasted_iota(jnp.int32, (128, 128), 1)
            psel = ((li % _L == 6) & (li // _L == mi // _L)).astype(jnp.float32)
            dv = 1.0 / jnp.dot(t[0], psel, preferred_element_type=jnp.float32)
            outs[1][...] = dv
        else:
            dv = dinv_ref[...]
        for cp in range(Cp):
            acc = jnp.zeros((rb, 128), jnp.float32)
            for c in range(C):
                acc = acc + jnp.dot(t[c], w[c, cp],
                                    preferred_element_type=jnp.float32)
            o = jnp.maximum(acc * dv + b[cp:cp + 1, :], 0.0)
            if kind == "last":
                ri = pl.program_id(0) * rb + lax.broadcasted_iota(
                    jnp.int32, (rb, 1), 0)
                s = jnp.sum(jnp.where(ri < nreal, o, 0.0), axis=0,
                            keepdims=True)
                pid = pl.program_id(0)

                @pl.when(pid == 0)
                def _():
                    outs[0][cp:cp + 1, :] = jnp.zeros((1, 128), jnp.float32)

                outs[0][cp:cp + 1, :] += s
            else:
                outs[0][cp] = o

    return pl.pallas_call(
        body,
        grid=grid,
        in_specs=in_specs,
        out_specs=out_specs,
        out_shape=out_shape,
        compiler_params=pltpu.CompilerParams(
            dimension_semantics=("arbitrary",)),
    )


def kernel(x, edge_index, W1, b1, W2, b2, W3, b3, W4, b4):
    N, d0 = x.shape
    NN = ((N + 127) // 128) * 128 // 8 * 8
    NN = ((N * _L // 128 + 7) // 8) * 8 * 128 // _L
    ncap = ((N + 1 + 127) // 128) * 128
    ei = jnp.asarray(edge_index, jnp.int32)
    src, dst = ei[0], ei[1]
    E = int(src.shape[0])
    epad = -(-E // (_NW * _EB)) * (_NW * _EB)
    pad = epad + _EB - E
    src = jnp.concatenate([src, jnp.zeros((pad,), jnp.int32)])
    dst = jnp.concatenate([dst, jnp.full((pad,), N, jnp.int32)])

    x_pad = jnp.concatenate(
        [jnp.pad(x.astype(jnp.float32), ((0, NN - N), (0, 0))),
         jnp.ones((NN, 1), jnp.float32),
         jnp.zeros((NN, _L - d0 - 1), jnp.float32)], axis=1)
    w1p = jnp.zeros((_L, W1.shape[1]), jnp.float32).at[:d0].set(W1)

    def wbig(w):
        ci, co = w.shape
        wv = w.reshape(ci // _L, _L, co // _L, _L)
        return jnp.einsum("ab,ckdj->cdakbj", jnp.eye(8, dtype=w.dtype),
                          wv).reshape(ci // _L, co // _L, 128, 128)

    def bpk(b):
        return jnp.tile(b.reshape(-1, _L), (1, 8))

    nrow = NN * _L // 128
    h = x_pad.reshape(1, nrow, 128)

    parts = _sc_agg_fn(1, N, NN, epad)(h.reshape(NN, _L), src, dst)
    parts = parts.reshape(_NC, 1, ncap * _L // 128, 128)
    h, dinv = _tc_layer_fn(1, 1, N, NN, "first")(
        parts, h, wbig(w1p), bpk(b1))

    parts = _sc_agg_fn(1, N, NN, epad)(h.reshape(NN, _L), src, dst)
    parts = parts.reshape(_NC, 1, ncap * _L // 128, 128)
    h = _tc_layer_fn(1, 2, N, NN, "mid")(parts, h, dinv, wbig(W2), bpk(b2))

    parts = _sc_agg_fn(2, N, NN, epad)(h.reshape(2 * NN, _L), src, dst)
    parts = parts.reshape(_NC, 2, ncap * _L // 128, 128)
    h = _tc_layer_fn(2, 4, N, NN, "mid")(parts, h, dinv, wbig(W3), bpk(b3))

    parts = _sc_agg_fn(4, N, NN, epad)(h.reshape(4 * NN, _L), src, dst)
    parts = parts.reshape(_NC, 4, ncap * _L // 128, 128)
    acc = _tc_layer_fn(4, 8, N, NN, "last")(parts, h, dinv, wbig(W4), bpk(b4))

    return (acc.reshape(8, 8, _L).sum(axis=1).reshape(-1) / N)


# speedup vs baseline: 21.2905x; 1.1869x over previous
import functools

import jax
import jax.numpy as jnp
from jax import lax
from jax.experimental import pallas as pl
from jax.experimental.pallas import tpu as pltpu
from jax.experimental.pallas import tpu_sc as plsc

_NC = 2
_NS = 16
_NW = _NC * _NS
_L = 16
_EB = 800


def _row_block(n):
    for r in (2000, 2500, 1000, 800, 500, 400, 250, 200, 125, 100, 50, 25, 8, 5, 4, 2, 1):
        if n % r == 0:
            return r
    return 1


@functools.cache
def _sc_agg_fn(C, N, NN, E):
    ncap = ((N + 1 + 127) // 128) * 128
    nslice = ncap // _NS
    epw = E // _NW
    nblk = epw // _EB
    assert epw % _EB == 0 and E % _NW == 0
    zch = []
    off = 0
    while off < nslice:
        sz = min(_EB, nslice - off)
        zch.append((off, sz))
        off += sz

    mesh = plsc.VectorSubcoreMesh(core_axis_name="c", subcore_axis_name="s")

    @functools.partial(
        pl.kernel,
        out_type=jax.ShapeDtypeStruct((_NC, C, ncap, _L), jnp.float32),
        mesh=mesh,
        scratch_types=[
            pltpu.VMEM_SHARED((ncap, _L), jnp.float32),
            pltpu.VMEM((_EB,), jnp.int32),
            pltpu.VMEM((_EB,), jnp.int32),
            pltpu.VMEM((_EB, _L), jnp.float32),
            pltpu.VMEM((_EB,), jnp.int32),
            pltpu.VMEM((_EB,), jnp.int32),
            pltpu.VMEM((_EB, _L), jnp.float32),
            pltpu.SemaphoreType.DMA,
            pltpu.SemaphoreType.DMA,
            pltpu.SemaphoreType.DMA,
            pltpu.SemaphoreType.DMA,
            pltpu.SemaphoreType.DMA,
            pltpu.SemaphoreType.DMA,
        ],
        compiler_params=pltpu.CompilerParams(use_tc_tiling_on_sc=False),
    )
    def agg(h_ref, src_ref, dst_ref, out_ref, acc,
            srcb0, dstb0, rowb0, srcb1, dstb1, rowb1,
            isem0, isem1, gsem0, gsem1, ssem0, ssem1):
        cid = lax.axis_index("c")
        sid = lax.axis_index("s")
        wid = sid * _NC + cid
        base = sid * nslice
        bufs = ((srcb0, dstb0, rowb0, isem0, gsem0, ssem0),
                (srcb1, dstb1, rowb1, isem1, gsem1, ssem1))

        def _idx_start(j, buf):
            srcb, dstb, _, isem, _, _ = buf
            e0 = wid * epw + j * _EB
            pltpu.async_copy(src_ref.at[pl.ds(e0, _EB)], srcb, isem)
            pltpu.async_copy(dst_ref.at[pl.ds(e0, _EB)], dstb, isem)

        def _idx_wait(buf):
            srcb, dstb, _, isem, _, _ = buf
            pltpu.make_async_copy(src_ref.at[pl.ds(0, _EB)], srcb, isem).wait()
            pltpu.make_async_copy(dst_ref.at[pl.ds(0, _EB)], dstb, isem).wait()

        def _gather_start(buf, c):
            srcb, _, rowb, _, gsem, _ = buf
            if c > 0:
                def _addo(i, cc):
                    srcb[pl.ds(i * _L, _L)] = srcb[pl.ds(i * _L, _L)] + (c * NN)
                    return cc

                lax.fori_loop(0, _EB // _L, _addo, 0)
            pltpu.async_copy(h_ref.at[srcb], rowb, gsem)

        def _gather_wait(buf):
            srcb, _, rowb, _, gsem, _ = buf
            pltpu.make_async_copy(h_ref.at[srcb], rowb, gsem).wait()

        def _scatter_start(buf):
            _, dstb, rowb, _, _, ssem = buf
            pltpu.async_copy(rowb, acc.at[dstb], ssem, add=True)

        def _scatter_wait(buf):
            _, dstb, rowb, _, _, ssem = buf
            pltpu.make_async_copy(rowb, acc.at[dstb], ssem).wait()

        for c in range(C):
            def _zero(i, carry):
                rowb0[i, :] = jnp.zeros((_L,), jnp.float32)
                return carry

            lax.fori_loop(0, _EB, _zero, 0)
            for zoff, zsz in zch:
                pltpu.sync_copy(rowb0.at[pl.ds(0, zsz)], acc.at[pl.ds(base + zoff, zsz)])
            plsc.subcore_barrier()

            _idx_start(0, bufs[0])
            _idx_start(1, bufs[1])
            _idx_wait(bufs[0])
            _gather_start(bufs[0], c)

            def _pair(k, carry):
                j = 2 * k
                _gather_wait(bufs[0])
                _idx_start(j + 2, bufs[0])

                @pl.when(k > 0)
                def _():
                    _scatter_wait(bufs[1])

                _idx_wait(bufs[1])
                _gather_start(bufs[1], c)
                _scatter_start(bufs[0])

                _gather_wait(bufs[1])
                _idx_start(j + 3, bufs[1])
                _scatter_wait(bufs[0])
                _idx_wait(bufs[0])
                _gather_start(bufs[0], c)
                _scatter_start(bufs[1])
                return carry

            lax.fori_loop(0, (nblk - 1) // 2, _pair, 0)
            _gather_wait(bufs[0])
            _scatter_wait(bufs[1])
            _scatter_start(bufs[0])
            _scatter_wait(bufs[0])
            _idx_wait(bufs[1])
            plsc.subcore_barrier()
            pltpu.sync_copy(acc.at[pl.ds(base, nslice)],
                            out_ref.at[cid, c, pl.ds(base, nslice)])

    return agg


@functools.cache
def _tc_layer_fn(C, Cp, N, NN, kind):
    ncap = ((N + 1 + 127) // 128) * 128
    rb = 256
    nreal = N * _L // 128
    nrow = NN * _L // 128
    grid = (-(-nreal // rb),)

    in_specs = [
        pl.BlockSpec((_NC, C, rb, 128), lambda i: (0, 0, i, 0)),
        pl.BlockSpec((C, rb, 128), lambda i: (0, i, 0)),
    ]
    if kind != "first":
        in_specs.append(pl.BlockSpec((rb, 128), lambda i: (i, 0)))
    in_specs.append(pl.BlockSpec((C, Cp, 128, 128), lambda i: (0, 0, 0, 0)))
    in_specs.append(pl.BlockSpec((Cp, 128), lambda i: (0, 0)))

    if kind == "first":
        out_shape = [
            jax.ShapeDtypeStruct((Cp, nrow, 128), jnp.float32),
            jax.ShapeDtypeStruct((nrow, 128), jnp.float32),
        ]
        out_specs = [
            pl.BlockSpec((Cp, rb, 128), lambda i: (0, i, 0)),
            pl.BlockSpec((rb, 128), lambda i: (i, 0)),
        ]
    elif kind == "mid":
        out_shape = jax.ShapeDtypeStruct((Cp, nrow, 128), jnp.float32)
        out_specs = pl.BlockSpec((Cp, rb, 128), lambda i: (0, i, 0))
    else:
        out_shape = jax.ShapeDtypeStruct((Cp, 128), jnp.float32)
        out_specs = pl.BlockSpec((Cp, 128), lambda i: (0, 0))

    def body(*refs):
        if kind == "first":
            parts, h, w, b = refs[:4]
            outs = refs[4:]
        else:
            parts, h, dinv_ref, w, b = refs[:5]
            outs = refs[5:]
        t = [parts[0, c] + parts[1, c] + h[c] for c in range(C)]
        if kind == "first":
            li = lax.broadcasted_iota(jnp.int32, (128, 128), 0)
            mi = lax.broadcasted_iota(jnp.int32, (128, 128), 1)
            psel = ((li % _L == 6) & (li // _L == mi // _L)).astype(jnp.float32)
            dv = 1.0 / jnp.dot(t[0], psel, preferred_element_type=jnp.float32)
            outs[1][...] = dv
        else:
            dv = dinv_ref[...]
        for cp in range(Cp):
            acc = jnp.zeros((rb, 128), jnp.float32)
            for c in range(C):
                acc = acc + jnp.dot(t[c], w[c, cp],
                                    preferred_element_type=jnp.float32)
            o = jnp.maximum(acc * dv + b[cp:cp + 1, :], 0.0)
            if kind == "last":
                ri = pl.program_id(0) * rb + lax.broadcasted_iota(
                    jnp.int32, (rb, 1), 0)
                s = jnp.sum(jnp.where(ri < nreal, o, 0.0), axis=0,
                            keepdims=True)
                pid = pl.program_id(0)

                @pl.when(pid == 0)
                def _():
                    outs[0][cp:cp + 1, :] = jnp.zeros((1, 128), jnp.float32)

                outs[0][cp:cp + 1, :] += s
            else:
                outs[0][cp] = o

    return pl.pallas_call(
        body,
        grid=grid,
        in_specs=in_specs,
        out_specs=out_specs,
        out_shape=out_shape,
        compiler_params=pltpu.CompilerParams(
            dimension_semantics=("arbitrary",)),
    )


def kernel(x, edge_index, W1, b1, W2, b2, W3, b3, W4, b4):
    N, d0 = x.shape
    NN = ((N + 127) // 128) * 128 // 8 * 8
    NN = ((N * _L // 128 + 7) // 8) * 8 * 128 // _L
    ncap = ((N + 1 + 127) // 128) * 128
    ei = jnp.asarray(edge_index, jnp.int32)
    src, dst = ei[0], ei[1]
    E = int(src.shape[0])
    epad = -(-E // (_NW * _EB)) * (_NW * _EB)
    if (epad // (_NW * _EB)) % 2 == 0:
        epad += _NW * _EB
    pad = epad + _EB - E
    ar = jnp.arange(pad, dtype=jnp.int32)
    src = jnp.concatenate([src, ar % N])
    dst = jnp.concatenate([dst, N + (ar % 64)])

    x_pad = jnp.concatenate(
        [jnp.pad(x.astype(jnp.float32), ((0, NN - N), (0, 0))),
         jnp.ones((NN, 1), jnp.float32),
         jnp.zeros((NN, _L - d0 - 1), jnp.float32)], axis=1)
    w1p = jnp.zeros((_L, W1.shape[1]), jnp.float32).at[:d0].set(W1)

    def wbig(w):
        ci, co = w.shape
        wv = w.reshape(ci // _L, _L, co // _L, _L)
        return jnp.einsum("ab,ckdj->cdakbj", jnp.eye(8, dtype=w.dtype),
                          wv).reshape(ci // _L, co // _L, 128, 128)

    def bpk(b):
        return jnp.tile(b.reshape(-1, _L), (1, 8))

    nrow = NN * _L // 128
    h = x_pad.reshape(1, nrow, 128)

    parts = _sc_agg_fn(1, N, NN, epad)(h.reshape(NN, _L), src, dst)
    parts = parts.reshape(_NC, 1, ncap * _L // 128, 128)
    h, dinv = _tc_layer_fn(1, 1, N, NN, "first")(
        parts, h, wbig(w1p), bpk(b1))

    parts = _sc_agg_fn(1, N, NN, epad)(h.reshape(NN, _L), src, dst)
    parts = parts.reshape(_NC, 1, ncap * _L // 128, 128)
    h = _tc_layer_fn(1, 2, N, NN, "mid")(parts, h, dinv, wbig(W2), bpk(b2))

    parts = _sc_agg_fn(2, N, NN, epad)(h.reshape(2 * NN, _L), src, dst)
    parts = parts.reshape(_NC, 2, ncap * _L // 128, 128)
    h = _tc_layer_fn(2, 4, N, NN, "mid")(parts, h, dinv, wbig(W3), bpk(b3))

    parts = _sc_agg_fn(4, N, NN, epad)(h.reshape(4 * NN, _L), src, dst)
    parts = parts.reshape(_NC, 4, ncap * _L // 128, 128)
    acc = _tc_layer_fn(4, 8, N, NN, "last")(parts, h, dinv, wbig(W4), bpk(b4))

    return (acc.reshape(8, 8, _L).sum(axis=1).reshape(-1) / N)
